# baseline (speedup 1.0000x reference)
import functools

import jax
import jax.numpy as jnp
from jax import lax
from jax.experimental import pallas as pl
from jax.experimental.pallas import tpu as pltpu
from jax.experimental.pallas import tpu_sc as plsc

BUCKET = 128
TOPK = 64
NW = 32


def _matmul_body(x_ref, q_ref, sim_ref, m_ref, qt_ref=None):
    sim = jnp.dot(x_ref[...], q_ref[...], preferred_element_type=jnp.float32)
    mrows, ncols = sim.shape
    nb = ncols // BUCKET
    sim3 = sim.reshape(mrows, nb, BUCKET)
    sim_ref[...] = sim3
    m_ref[0] = jnp.max(sim3, axis=-1)

    if qt_ref is not None:
        @pl.when(pl.program_id(1) == 0)
        def _():
            qt_ref[...] = jnp.swapaxes(q_ref[...], 0, 1)


def _sim_and_bucketmax(xf, queue, emit_qt=True):
    M, D = xf.shape
    K = queue.shape[1]
    NC = 2048
    MR = 256
    NB = NC // BUCKET
    grid = (K // NC, M // MR)
    return pl.pallas_call(
        _matmul_body,
        grid=grid,
        in_specs=[
            pl.BlockSpec((MR, D), lambda j, i: (i, 0)),
            pl.BlockSpec((D, NC), lambda j, i: (0, j)),
        ],
        out_specs=[
            pl.BlockSpec((MR, NB, BUCKET), lambda j, i: (i, j, 0)),
            pl.BlockSpec((1, MR, NB), lambda j, i: (j, i, 0)),
        ] + ([pl.BlockSpec((NC, D), lambda j, i: (j, 0))] if emit_qt else []),
        out_shape=[
            jax.ShapeDtypeStruct((M, K // BUCKET, BUCKET), jnp.float32),
            jax.ShapeDtypeStruct((K // NC, M, NB), jnp.float32),
        ] + ([jax.ShapeDtypeStruct((K, D), jnp.float32)] if emit_qt else []),
    )(xf, queue)


def _make_selector(R, KB, n_sim_rows):
    rows_per_w = R // NW
    NV = KB // 16
    mesh = plsc.VectorSubcoreMesh(core_axis_name="c", subcore_axis_name="s")

    D = 512
    @functools.partial(
        pl.kernel,
        out_type=(jax.ShapeDtypeStruct((R, TOPK), jnp.int32),
                  jax.ShapeDtypeStruct((R * TOPK, D), jnp.float32)),
        mesh=mesh,
        compiler_params=pltpu.CompilerParams(
            needs_layout_passes=False, use_tc_tiling_on_sc=False),
        scratch_types=[
            pltpu.VMEM((KB,), jnp.float32),
            pltpu.VMEM((KB + 16,), jnp.int32),
            pltpu.VMEM((KB + 96,), jnp.float32),
            pltpu.VMEM((KB + 16,), jnp.int32),
            pltpu.VMEM((KB + 16,), jnp.float32),
            pltpu.VMEM((KB, BUCKET), jnp.float32),
            pltpu.VMEM((TOPK,), jnp.int32),
            pltpu.VMEM((TOPK, D), jnp.float32),
            pltpu.SemaphoreType.DMA,
            pltpu.SemaphoreType.DMA,
            pltpu.SemaphoreType.DMA,
        ],
    )
    def sel(sim_rows, m_hbm, qt_hbm, out_hbm, samp_hbm, m_v, cid_v, cmax_v,
            cid2_v, cmax2_v, cval_v, out_v, fbuf, sem, semg, semo):
        wid = lax.axis_index("s") * 2 + lax.axis_index("c")
        iota = lax.iota(jnp.int32, 16)
        ninf = jnp.float32(-jnp.inf)
        ninf16 = jnp.full((16,), ninf, jnp.float32)

        def init_body(j, c):
            cid2_v[pl.ds(16 * j, 16)] = 16 * j + iota
            return c

        lax.fori_loop(0, (KB + 16) // 16, init_body, 0)

        def row_body(t, carry):
            r = wid * rows_per_w + t
            pltpu.sync_copy(m_hbm.at[r], m_v)

            def t0_body(j, acc):
                g0, g1, g2, g3 = acc
                b = 64 * j
                g0 = jnp.maximum(g0, m_v[pl.ds(b, 16)])
                g1 = jnp.maximum(g1, m_v[pl.ds(b + 16, 16)])
                g2 = jnp.maximum(g2, m_v[pl.ds(b + 32, 16)])
                g3 = jnp.maximum(g3, m_v[pl.ds(b + 48, 16)])
                return (g0, g1, g2, g3)

            g0, g1, g2, g3 = lax.fori_loop(
                0, KB // 64, t0_body, (ninf16, ninf16, ninf16, ninf16))
            t0 = jnp.min(jnp.minimum(jnp.minimum(g0, g1), jnp.minimum(g2, g3)))

            def scan_body(j, cnt):
                v = m_v[pl.ds(16 * j, 16)]
                msk = v >= t0
                ids = r * KB + 16 * j + iota
                plsc.store_compressed(cid_v.at[pl.ds(cnt, 16)], ids, mask=msk)
                plsc.store_compressed(cmax_v.at[pl.ds(cnt, 16)], v, mask=msk)
                return cnt + jnp.sum(msk.astype(jnp.int32))

            cnt = lax.fori_loop(0, NV, scan_body, jnp.int32(0))
            for p in range(6):
                cmax_v[pl.ds(cnt + 16 * p, 16)] = ninf16
            nv = (cnt + 15) // 16

            def t1_body(j, acc):
                g0, g1, g2, g3 = acc
                b = 64 * j
                g0 = jnp.maximum(g0, cmax_v[pl.ds(b, 16)])
                g1 = jnp.maximum(g1, cmax_v[pl.ds(b + 16, 16)])
                g2 = jnp.maximum(g2, cmax_v[pl.ds(b + 32, 16)])
                g3 = jnp.maximum(g3, cmax_v[pl.ds(b + 48, 16)])
                return (g0, g1, g2, g3)

            g0, g1, g2, g3 = lax.fori_loop(
                0, (nv + 3) // 4, t1_body, (ninf16, ninf16, ninf16, ninf16))
            t1 = jnp.min(jnp.minimum(jnp.minimum(g0, g1), jnp.minimum(g2, g3)))

            def refilter(j, cnt2):
                v = cmax_v[pl.ds(16 * j, 16)]
                ids = cid_v[pl.ds(16 * j, 16)]
                msk = v >= t1
                plsc.store_compressed(cid2_v.at[pl.ds(cnt2, 16)], ids, mask=msk)
                plsc.store_compressed(cmax2_v.at[pl.ds(cnt2, 16)], v, mask=msk)
                return cnt2 + jnp.sum(msk.astype(jnp.int32))

            cnt2 = lax.fori_loop(0, nv, refilter, jnp.int32(0))
            cmax2_v[pl.ds(cnt2, 16)] = ninf16
            nv2 = (cnt2 + 15) // 16

            hs = []
            for ck in range(KB // 32):
                @pl.when(cnt2 > 32 * ck)
                def _(ck=ck):
                    hs.append(pltpu.async_copy(
                        sim_rows.at[cid2_v.at[pl.ds(32 * ck, 32)]],
                        cval_v.at[pl.ds(32 * ck, 32)],
                        sem,
                    ))
            for ck in range(KB // 32):
                @pl.when(cnt2 > 32 * ck)
                def _(ck=ck):
                    hs[ck].wait()

            def vm_build(j, vm):
                m = jnp.max(cmax2_v[pl.ds(16 * j, 16)])
                return jnp.where(iota == j, m, vm)

            vm0 = lax.fori_loop(0, nv2, vm_build, ninf16)

            def ext_inner(ii, carry):
                ovec, vm = carry
                best = jnp.max(vm)
                bj = jnp.min(jnp.where(vm == best, iota, 16))
                cm = cmax2_v[pl.ds(16 * bj, 16)]
                l = jnp.min(jnp.where(cm == best, iota, 16))
                slot = 16 * bj + l
                ids = cid2_v[pl.ds(16 * bj, 16)]
                bid = jnp.max(jnp.where(iota == l, ids, 0))
                b_local = bid - r * KB
                ksel = jnp.int32(BUCKET // 16)
                for k in range(BUCKET // 16):
                    wk = cval_v[slot, pl.ds(16 * k, 16)]
                    hit = jnp.sum((wk == best).astype(jnp.int32)) > 0
                    ksel = jnp.where(jnp.logical_and(hit, ksel == BUCKET // 16),
                                     k, ksel)
                w = cval_v[slot, pl.ds(16 * ksel, 16)]
                el = jnp.min(jnp.where(w == best, iota, 16))
                col = b_local * BUCKET + 16 * ksel + el
                cval_v[slot, pl.ds(16 * ksel, 16)] = jnp.where(
                    iota == el, ninf, w)
                newm = ninf
                for k in range(BUCKET // 16):
                    newm = jnp.maximum(newm,
                                       jnp.max(cval_v[slot, pl.ds(16 * k, 16)]))
                cm2 = jnp.where(iota == l, newm, cm)
                cmax2_v[pl.ds(16 * bj, 16)] = cm2
                vm = jnp.where(iota == bj, jnp.max(cm2), vm)
                return (jnp.where(iota == (ii % 16), col, ovec), vm)

            for oi in range(TOPK // 16):
                ovec, vm0 = lax.fori_loop(16 * oi, 16 * (oi + 1), ext_inner,
                                          (jnp.zeros((16,), jnp.int32), vm0))
                out_v[pl.ds(16 * oi, 16)] = ovec

            @pl.when(t > 0)
            def _():
                pltpu.make_async_copy(
                    fbuf, samp_hbm.at[pl.ds(0, TOPK)], semo).wait()
            hg = pltpu.async_copy(qt_hbm.at[out_v], fbuf, semg)
            pltpu.sync_copy(out_v, out_hbm.at[r])
            hg.wait()
            pltpu.async_copy(fbuf, samp_hbm.at[pl.ds(r * TOPK, TOPK)], semo)
            return carry

        lax.fori_loop(0, rows_per_w, row_body, 0)
        pltpu.make_async_copy(fbuf, samp_hbm.at[pl.ds(0, TOPK)], semo).wait()

    return sel


def _make_gather(Kq, D, Btot):
    bpw = Btot // NW
    CH = 64
    NCH = bpw // CH
    mesh = plsc.VectorSubcoreMesh(core_axis_name="c", subcore_axis_name="s")

    @functools.partial(
        pl.kernel,
        out_type=jax.ShapeDtypeStruct((Btot, D), jnp.float32),
        mesh=mesh,
        compiler_params=pltpu.CompilerParams(needs_layout_passes=False),
        scratch_types=[
            pltpu.VMEM((bpw,), jnp.int32),
            pltpu.VMEM((CH, D), jnp.float32),
            pltpu.VMEM((CH, D), jnp.float32),
            pltpu.SemaphoreType.DMA,
            pltpu.SemaphoreType.DMA,
            pltpu.SemaphoreType.DMA,
            pltpu.SemaphoreType.DMA,
        ],
    )
    def gat(qt_hbm, idx_hbm, out_hbm, idx_v, buf0, buf1, s0, s1, so0, so1):
        wid = lax.axis_index("s") * 2 + lax.axis_index("c")
        base = wid * bpw
        pltpu.sync_copy(idx_hbm.at[pl.ds(base, bpw)], idx_v)
        bufs = (buf0, buf1)
        sin = (s0, s1)
        sout = (so0, so1)
        hin = [None] * NCH
        hout = [None] * NCH
        hin[0] = pltpu.async_copy(
            qt_hbm.at[idx_v.at[pl.ds(0, CH)]], buf0, s0)
        for ch in range(NCH):
            if ch + 1 < NCH:
                if ch >= 1:
                    hout[ch - 1].wait()
                hin[ch + 1] = pltpu.async_copy(
                    qt_hbm.at[idx_v.at[pl.ds((ch + 1) * CH, CH)]],
                    bufs[(ch + 1) % 2], sin[(ch + 1) % 2])
            hin[ch].wait()
            hout[ch] = pltpu.async_copy(
                bufs[ch % 2], out_hbm.at[pl.ds(base + ch * CH, CH)],
                sout[ch % 2])
        hout[NCH - 2].wait()
        hout[NCH - 1].wait()

    return gat


def kernel(x, topk, queue):
    B, N, D = x.shape
    K = queue.shape[1]
    R = B * N
    KB = K // BUCKET
    xf = x.reshape(R, D)
    H = R // 2
    sel = _make_selector(H, KB, H * KB)
    ind_halves, samp_halves = [], []
    qT = None
    for h in range(2):
        xh = xf[h * H:(h + 1) * H]
        outs = _sim_and_bucketmax(xh, queue, emit_qt=(h == 0))
        sim3, bmax3 = outs[0], outs[1]
        if h == 0:
            qT = outs[2]
        bmax = jnp.transpose(bmax3, (1, 0, 2)).reshape(H, KB)
        sim_rows = sim3.reshape(H * KB, BUCKET)
        inds_h, samp_h = sel(sim_rows, bmax, qT)
        ind_halves.append(inds_h)
        samp_halves.append(samp_h)
    inds = jnp.concatenate(ind_halves, axis=0)
    sampled = jnp.concatenate(samp_halves, axis=0)
    return (sampled.reshape(B, N, TOPK, D), inds.reshape(B, N, TOPK))


# speedup vs baseline: 1.4418x; 1.4418x over previous
import functools

import jax
import jax.numpy as jnp
from jax import lax
from jax.experimental import pallas as pl
from jax.experimental.pallas import tpu as pltpu
from jax.experimental.pallas import tpu_sc as plsc

BUCKET = 128
TOPK = 64
NW = 32


def _matmul_body(x_ref, q_ref, sim_ref, m_ref, qt_ref):
    sim = jnp.dot(x_ref[...], q_ref[...], preferred_element_type=jnp.float32)
    mrows, ncols = sim.shape
    nb = ncols // BUCKET
    sim3 = sim.reshape(mrows, nb, BUCKET)
    sim_ref[...] = sim3
    m_ref[0] = jnp.max(sim3, axis=-1)

    @pl.when(pl.program_id(1) == 0)
    def _():
        qt_ref[...] = jnp.swapaxes(q_ref[...], 0, 1)


def _sim_and_bucketmax(xf, queue):
    M, D = xf.shape
    K = queue.shape[1]
    NC = 2048
    MR = 256
    NB = NC // BUCKET
    grid = (K // NC, M // MR)
    return pl.pallas_call(
        _matmul_body,
        grid=grid,
        in_specs=[
            pl.BlockSpec((MR, D), lambda j, i: (i, 0)),
            pl.BlockSpec((D, NC), lambda j, i: (0, j)),
        ],
        out_specs=[
            pl.BlockSpec((MR, NB, BUCKET), lambda j, i: (i, j, 0)),
            pl.BlockSpec((1, MR, NB), lambda j, i: (j, i, 0)),
            pl.BlockSpec((NC, D), lambda j, i: (j, 0)),
        ],
        out_shape=[
            jax.ShapeDtypeStruct((M, K // BUCKET, BUCKET), jnp.float32),
            jax.ShapeDtypeStruct((K // NC, M, NB), jnp.float32),
            jax.ShapeDtypeStruct((K, D), jnp.float32),
        ],
    )(xf, queue)


def _make_selector(R, KB, n_sim_rows):
    rows_per_w = R // NW
    NV = KB // 16
    mesh = plsc.VectorSubcoreMesh(core_axis_name="c", subcore_axis_name="s")

    @functools.partial(
        pl.kernel,
        out_type=jax.ShapeDtypeStruct((R, TOPK), jnp.int32),
        mesh=mesh,
        compiler_params=pltpu.CompilerParams(
            needs_layout_passes=False, use_tc_tiling_on_sc=False),
        scratch_types=[
            pltpu.VMEM((KB,), jnp.float32),
            pltpu.VMEM((KB + 16,), jnp.int32),
            pltpu.VMEM((KB + 96,), jnp.float32),
            pltpu.VMEM((KB + 16,), jnp.int32),
            pltpu.VMEM((KB + 16,), jnp.float32),
            pltpu.VMEM((KB, BUCKET), jnp.float32),
            pltpu.VMEM((TOPK,), jnp.int32),
            pltpu.SemaphoreType.DMA,
        ],
    )
    def sel(sim_rows, m_hbm, out_hbm, m_v, cid_v, cmax_v, cid2_v, cmax2_v,
            cval_v, out_v, sem):
        wid = lax.axis_index("s") * 2 + lax.axis_index("c")
        iota = lax.iota(jnp.int32, 16)
        ninf = jnp.float32(-jnp.inf)
        ninf16 = jnp.full((16,), ninf, jnp.float32)

        def init_body(j, c):
            cid2_v[pl.ds(16 * j, 16)] = 16 * j + iota
            return c

        lax.fori_loop(0, (KB + 16) // 16, init_body, 0)

        def row_body(t, carry):
            r = wid * rows_per_w + t
            pltpu.sync_copy(m_hbm.at[r], m_v)

            def t0_body(j, acc):
                g0, g1, g2, g3 = acc
                b = 64 * j
                g0 = jnp.maximum(g0, m_v[pl.ds(b, 16)])
                g1 = jnp.maximum(g1, m_v[pl.ds(b + 16, 16)])
                g2 = jnp.maximum(g2, m_v[pl.ds(b + 32, 16)])
                g3 = jnp.maximum(g3, m_v[pl.ds(b + 48, 16)])
                return (g0, g1, g2, g3)

            g0, g1, g2, g3 = lax.fori_loop(
                0, KB // 64, t0_body, (ninf16, ninf16, ninf16, ninf16))
            t0 = jnp.min(jnp.minimum(jnp.minimum(g0, g1), jnp.minimum(g2, g3)))

            def scan_body(j, cnt):
                v = m_v[pl.ds(16 * j, 16)]
                msk = v >= t0
                ids = r * KB + 16 * j + iota
                plsc.store_compressed(cid_v.at[pl.ds(cnt, 16)], ids, mask=msk)
                plsc.store_compressed(cmax_v.at[pl.ds(cnt, 16)], v, mask=msk)
                return cnt + jnp.sum(msk.astype(jnp.int32))

            cnt = lax.fori_loop(0, NV, scan_body, jnp.int32(0))
            for p in range(6):
                cmax_v[pl.ds(cnt + 16 * p, 16)] = ninf16
            nv = (cnt + 15) // 16

            def t1_body(j, acc):
                g0, g1, g2, g3 = acc
                b = 64 * j
                g0 = jnp.maximum(g0, cmax_v[pl.ds(b, 16)])
                g1 = jnp.maximum(g1, cmax_v[pl.ds(b + 16, 16)])
                g2 = jnp.maximum(g2, cmax_v[pl.ds(b + 32, 16)])
                g3 = jnp.maximum(g3, cmax_v[pl.ds(b + 48, 16)])
                return (g0, g1, g2, g3)

            g0, g1, g2, g3 = lax.fori_loop(
                0, (nv + 3) // 4, t1_body, (ninf16, ninf16, ninf16, ninf16))
            t1 = jnp.min(jnp.minimum(jnp.minimum(g0, g1), jnp.minimum(g2, g3)))

            def refilter(j, cnt2):
                v = cmax_v[pl.ds(16 * j, 16)]
                ids = cid_v[pl.ds(16 * j, 16)]
                msk = v >= t1
                plsc.store_compressed(cid2_v.at[pl.ds(cnt2, 16)], ids, mask=msk)
                plsc.store_compressed(cmax2_v.at[pl.ds(cnt2, 16)], v, mask=msk)
                return cnt2 + jnp.sum(msk.astype(jnp.int32))

            cnt2 = lax.fori_loop(0, nv, refilter, jnp.int32(0))
            cmax2_v[pl.ds(cnt2, 16)] = ninf16
            nv2 = (cnt2 + 15) // 16

            hs = []
            for ck in range(KB // 32):
                @pl.when(cnt2 > 32 * ck)
                def _(ck=ck):
                    hs.append(pltpu.async_copy(
                        sim_rows.at[cid2_v.at[pl.ds(32 * ck, 32)]],
                        cval_v.at[pl.ds(32 * ck, 32)],
                        sem,
                    ))
            for ck in range(KB // 32):
                @pl.when(cnt2 > 32 * ck)
                def _(ck=ck):
                    hs[ck].wait()

            def vm_build(j, vm):
                m = jnp.max(cmax2_v[pl.ds(16 * j, 16)])
                return jnp.where(iota == j, m, vm)

            vm0 = lax.fori_loop(0, nv2, vm_build, ninf16)

            def ext_inner(ii, carry):
                ovec, vm = carry
                best = jnp.max(vm)
                bj = jnp.min(jnp.where(vm == best, iota, 16))
                cm = cmax2_v[pl.ds(16 * bj, 16)]
                l = jnp.min(jnp.where(cm == best, iota, 16))
                slot = 16 * bj + l
                ids = cid2_v[pl.ds(16 * bj, 16)]
                bid = jnp.max(jnp.where(iota == l, ids, 0))
                b_local = bid - r * KB
                ksel = jnp.int32(BUCKET // 16)
                for k in range(BUCKET // 16):
                    wk = cval_v[slot, pl.ds(16 * k, 16)]
                    hit = jnp.sum((wk == best).astype(jnp.int32)) > 0
                    ksel = jnp.where(jnp.logical_and(hit, ksel == BUCKET // 16),
                                     k, ksel)
                w = cval_v[slot, pl.ds(16 * ksel, 16)]
                el = jnp.min(jnp.where(w == best, iota, 16))
                col = b_local * BUCKET + 16 * ksel + el
                cval_v[slot, pl.ds(16 * ksel, 16)] = jnp.where(
                    iota == el, ninf, w)
                newm = ninf
                for k in range(BUCKET // 16):
                    newm = jnp.maximum(newm,
                                       jnp.max(cval_v[slot, pl.ds(16 * k, 16)]))
                cm2 = jnp.where(iota == l, newm, cm)
                cmax2_v[pl.ds(16 * bj, 16)] = cm2
                vm = jnp.where(iota == bj, jnp.max(cm2), vm)
                return (jnp.where(iota == (ii % 16), col, ovec), vm)

            for oi in range(TOPK // 16):
                ovec, vm0 = lax.fori_loop(16 * oi, 16 * (oi + 1), ext_inner,
                                          (jnp.zeros((16,), jnp.int32), vm0))
                out_v[pl.ds(16 * oi, 16)] = ovec

            pltpu.sync_copy(out_v, out_hbm.at[r])
            return carry

        lax.fori_loop(0, rows_per_w, row_body, 0)

    return sel


def _make_gather(Kq, D, Btot):
    bpw = Btot // NW
    CH = 64
    NCH = bpw // CH
    mesh = plsc.VectorSubcoreMesh(core_axis_name="c", subcore_axis_name="s")

    @functools.partial(
        pl.kernel,
        out_type=jax.ShapeDtypeStruct((Btot, D), jnp.float32),
        mesh=mesh,
        compiler_params=pltpu.CompilerParams(needs_layout_passes=False),
        scratch_types=[
            pltpu.VMEM((bpw,), jnp.int32),
            pltpu.VMEM((CH, D), jnp.float32),
            pltpu.VMEM((CH, D), jnp.float32),
            pltpu.SemaphoreType.DMA,
            pltpu.SemaphoreType.DMA,
            pltpu.SemaphoreType.DMA,
            pltpu.SemaphoreType.DMA,
        ],
    )
    def gat(qt_hbm, idx_hbm, out_hbm, idx_v, buf0, buf1, s0, s1, so0, so1):
        wid = lax.axis_index("s") * 2 + lax.axis_index("c")
        base = wid * bpw
        pltpu.sync_copy(idx_hbm.at[pl.ds(base, bpw)], idx_v)
        bufs = (buf0, buf1)
        sin = (s0, s1)
        sout = (so0, so1)
        hin = [None] * NCH
        hout = [None] * NCH
        hin[0] = pltpu.async_copy(
            qt_hbm.at[idx_v.at[pl.ds(0, CH)]], buf0, s0)
        for ch in range(NCH):
            if ch + 1 < NCH:
                if ch >= 1:
                    hout[ch - 1].wait()
                hin[ch + 1] = pltpu.async_copy(
                    qt_hbm.at[idx_v.at[pl.ds((ch + 1) * CH, CH)]],
                    bufs[(ch + 1) % 2], sin[(ch + 1) % 2])
            hin[ch].wait()
            hout[ch] = pltpu.async_copy(
                bufs[ch % 2], out_hbm.at[pl.ds(base + ch * CH, CH)],
                sout[ch % 2])
        hout[NCH - 2].wait()
        hout[NCH - 1].wait()

    return gat


def kernel(x, topk, queue):
    B, N, D = x.shape
    K = queue.shape[1]
    R = B * N
    KB = K // BUCKET
    xf = x.reshape(R, D)
    H = R // 2
    sel = _make_selector(H, KB, H * KB)
    halves = []
    qT = None
    for h in range(2):
        xh = xf[h * H:(h + 1) * H]
        sim3, bmax3, qTh = _sim_and_bucketmax(xh, queue)
        if h == 0:
            qT = qTh
        bmax = jnp.transpose(bmax3, (1, 0, 2)).reshape(H, KB)
        sim_rows = sim3.reshape(H * KB, BUCKET)
        halves.append(sel(sim_rows, bmax))
    inds = jnp.concatenate(halves, axis=0)
    flat = inds.reshape(-1)
    sampled = _make_gather(K, D, R * TOPK)(qT, flat)
    return (sampled.reshape(B, N, TOPK, D), inds.reshape(B, N, TOPK))
